# baseline (speedup 1.0000x reference)
import functools

import jax
import jax.numpy as jnp
from jax import lax
from jax.experimental import pallas as pl
from jax.experimental.pallas import tpu as pltpu
from jax.experimental.pallas import tpu_sc as plsc

B, C, H, W = 32, 768, 32, 32
N = H * W
R = B * C
K = 205
ALPHA = 0.7

NC, NS, L = 2, 16, 16
NW = NC * NS
RPW = R // NW
GROUP = 32
NGRP = RPW // GROUP
CH32 = N // (2 * L)
NBITS = 12

_DNUMS = lax.GatherDimensionNumbers(
    offset_dims=(), collapsed_slice_dims=(0,), start_index_map=(0,))


def _permute(v, p):
    return lax.gather(v, p[:, None], dimension_numbers=_DNUMS,
                      slice_sizes=(1,),
                      mode=lax.GatherScatterMode.PROMISE_IN_BOUNDS)


def _kernel_body(x_hbm, out_hbm, xbuf, kbuf, outbuf, sem0, sem1):
    wid = lax.axis_index("s") * NC + lax.axis_index("c")
    zero = jnp.zeros((L,), jnp.int32)
    one = jnp.ones((L,), jnp.int32)
    fzero = jnp.zeros((L,), jnp.float32)
    bzero = jnp.zeros((2 * L,), jnp.bfloat16)
    bone = jnp.ones((2 * L,), jnp.bfloat16)
    lanes = lax.iota(jnp.int32, L)
    perms = [lanes ^ sh for sh in (8, 4, 2, 1)]

    def allsum(v):
        for p in perms:
            v = v + _permute(v, p)
        return v

    def u2bits(u):
        return jnp.where(u >= 32768, u - 32768, 65535 - u)

    def u2f32(u):
        return plsc.bitcast(u2bits(u) << 16, jnp.float32)

    def u2bf(u):
        b = u2bits(u)
        return plsc.bitcast(b | (b << 16), jnp.bfloat16)

    GN = GROUP * N

    def copy_in(g, buf_i, sem):
        row0 = wid * RPW + g * GROUP
        return pltpu.make_async_copy(
            x_hbm.at[pl.ds(row0 * N, GN)],
            xbuf.at[pl.ds(buf_i * GN, GN)], sem)

    copy_in(0, 0, sem0).start()

    def group_body(g, carry):
        parity = g & 1

        @pl.when(parity == 0)
        def _():
            copy_in(g, 0, sem0).wait()

        @pl.when(parity == 1)
        def _():
            copy_in(g, 1, sem1).wait()

        @pl.when((g + 1 < NGRP) & (parity == 0))
        def _():
            copy_in(g + 1, 1, sem1).start()

        @pl.when((g + 1 < NGRP) & (parity == 1))
        def _():
            copy_in(g + 1, 0, sem0).start()

        boff = parity * GN

        def key_body(j, c):
            p = j >> 2
            e0 = (j & 3) * 16
            for u in range(16):
                e = e0 + u
                a = xbuf[pl.ds(boff + (2 * p) * N + e * L, L)]
                b = xbuf[pl.ds(boff + (2 * p + 1) * N + e * L, L)]
                pk = plsc.pack(a, b, format=plsc.PackFormat.INTERLEAVED)
                kbuf[pl.ds(p * N + e * L, L)] = plsc.bitcast(pk, jnp.int32)
            return c

        lax.fori_loop(0, GROUP * N // (32 * L), key_body, 0)

        def pair_body(p, ov):
            ovec, ovec2 = ov
            kbase = p * N

            def pk2bf(bitsA, bitsB):
                return plsc.bitcast(bitsA | (bitsB << 16), jnp.bfloat16)

            t1A = t1B = t2A = t2B = zero
            for i in range(NBITS):
                bitc = 32768 >> i
                c1A, c1B = t1A + bitc, t1B + bitc
                c2A, c2B = t2A + bitc, t2B + bitc
                cv1 = pk2bf(u2bits(c1A), u2bits(c1B))
                cv2 = pk2bf(u2bits(65535 - c2A), u2bits(65535 - c2B))

                def cbody(j, c, cv1=cv1, cv2=cv2):
                    c1a, c1b, c2a, c2b = c
                    for u in range(4):
                        v = plsc.bitcast(
                            kbuf[pl.ds(kbase + (j * 4 + u) * L, L)],
                            jnp.bfloat16)
                        i1 = jnp.where(v >= cv1, bone, bzero)
                        i2 = jnp.where(v <= cv2, bone, bzero)
                        if u % 2 == 0:
                            c1a = c1a + i1
                            c2a = c2a + i2
                        else:
                            c1b = c1b + i1
                            c2b = c2b + i2
                    return c1a, c1b, c2a, c2b

                c1a, c1b, c2a, c2b = lax.fori_loop(
                    0, N // (4 * L), cbody, (bzero, bzero, bzero, bzero))
                uA1, uB1 = plsc.unpack(c1a + c1b,
                                       format=plsc.PackFormat.INTERLEAVED)
                uA2, uB2 = plsc.unpack(c2a + c2b,
                                       format=plsc.PackFormat.INTERLEAVED)
                nA1, nB1 = allsum(uA1), allsum(uB1)
                nA2, nB2 = allsum(uA2), allsum(uB2)
                t1A = jnp.where(nA1 >= float(K), c1A, t1A)
                t1B = jnp.where(nB1 >= float(K), c1B, t1B)
                t2A = jnp.where(nA2 >= float(K), c2A, t2A)
                t2B = jnp.where(nB2 >= float(K), c2B, t2B)

            for rr, (t1, t2) in enumerate(((t1A, t2A), (t1B, t2B))):
                r = 2 * p + rr
                base = r * N
                bot = 65535 - t2

                val_top = 0.5 * (u2f32(t1) + u2f32(t1 + 15))
                val_bot = 0.5 * (u2f32(bot - 15) + u2f32(bot))
                ub = 0.5 * (u2f32(t1 + 15) + u2f32(t1 + 16))
                lb = 0.5 * (u2f32(bot - 16) + u2f32(bot - 15))

                def fbody(j, c, base=base, ub=ub, lb=lb):
                    cg, sg, cl, sl = c
                    for u in range(8):
                        xv = xbuf[pl.ds(boff + base + (j * 8 + u) * L, L)]
                        m1 = xv > ub
                        m2 = xv < lb
                        cg = cg + jnp.where(m1, one, zero)
                        sg = sg + jnp.where(m1, xv, fzero)
                        cl = cl + jnp.where(m2, one, zero)
                        sl = sl + jnp.where(m2, xv, fzero)
                    return cg, sg, cl, sl

                cg, sg, cl, sl = lax.fori_loop(
                    0, N // (8 * L), fbody, (zero, fzero, zero, fzero))

                ng = float(K) - allsum(cg.astype(jnp.float32))
                nl = float(K) - allsum(cl.astype(jnp.float32))
                sgv = allsum(sg)
                slv = allsum(sl)
                top_sum = sgv + ng * val_top
                bot_sum = slv + nl * val_bot
                outv = (top_sum * (1.0 / (2 * K))
                        + bot_sum * (ALPHA / (2 * K)))
                lo = p < 8
                msk = lanes == (r & 15)
                ovec = jnp.where(msk & lo, outv, ovec)
                ovec2 = jnp.where(msk & ~lo, outv, ovec2)
            return ovec, ovec2

        ovec, ovec2 = lax.fori_loop(
            0, GROUP // 2, pair_body, (fzero, fzero))
        outbuf[pl.ds(g * GROUP, 16)] = ovec
        outbuf[pl.ds(g * GROUP + 16, 16)] = ovec2
        return carry

    lax.fori_loop(0, NGRP, group_body, 0)
    pltpu.sync_copy(outbuf, out_hbm.at[pl.ds(wid * RPW, RPW)])


@jax.jit
def kernel(input):
    x = input.reshape(R * N)
    mesh = plsc.VectorSubcoreMesh(
        core_axis_name="c", subcore_axis_name="s",
        num_cores=NC, num_subcores=NS)
    out = pl.kernel(
        _kernel_body,
        out_type=jax.ShapeDtypeStruct((R,), jnp.float32),
        mesh=mesh,
        compiler_params=pltpu.CompilerParams(needs_layout_passes=False),
        scratch_types=[
            pltpu.VMEM((2 * GROUP * N,), jnp.float32),
            pltpu.VMEM((GROUP * N // 2,), jnp.int32),
            pltpu.VMEM((RPW,), jnp.float32),
            pltpu.SemaphoreType.DMA,
            pltpu.SemaphoreType.DMA,
        ],
    )(x)
    return out.reshape(B, C)


# speedup vs baseline: 1.0011x; 1.0011x over previous
import functools

import jax
import jax.numpy as jnp
from jax import lax
from jax.experimental import pallas as pl
from jax.experimental.pallas import tpu as pltpu
from jax.experimental.pallas import tpu_sc as plsc

B, C, H, W = 32, 768, 32, 32
N = H * W
R = B * C
K = 205
ALPHA = 0.7

NC, NS, L = 2, 16, 16
NW = NC * NS
RPW = R // NW
GROUP = 16
NGRP = RPW // GROUP
CH32 = N // (2 * L)
NBITS = 12

_DNUMS = lax.GatherDimensionNumbers(
    offset_dims=(), collapsed_slice_dims=(0,), start_index_map=(0,))


def _permute(v, p):
    return lax.gather(v, p[:, None], dimension_numbers=_DNUMS,
                      slice_sizes=(1,),
                      mode=lax.GatherScatterMode.PROMISE_IN_BOUNDS)


def _kernel_body(x_hbm, out_hbm, xbuf, kbuf, outbuf, sem0, sem1):
    wid = lax.axis_index("s") * NC + lax.axis_index("c")
    zero = jnp.zeros((L,), jnp.int32)
    one = jnp.ones((L,), jnp.int32)
    fzero = jnp.zeros((L,), jnp.float32)
    bzero = jnp.zeros((2 * L,), jnp.bfloat16)
    bone = jnp.ones((2 * L,), jnp.bfloat16)
    lanes = lax.iota(jnp.int32, L)
    perms = [lanes ^ sh for sh in (8, 4, 2, 1)]

    def allsum(v):
        for p in perms:
            v = v + _permute(v, p)
        return v

    def u2bits(u):
        return jnp.where(u >= 32768, u - 32768, 65535 - u)

    def u2f32(u):
        return plsc.bitcast(u2bits(u) << 16, jnp.float32)

    def u2bf(u):
        b = u2bits(u)
        return plsc.bitcast(b | (b << 16), jnp.bfloat16)

    GN = GROUP * N

    def copy_in(g, buf_i, sem):
        row0 = wid * RPW + g * GROUP
        return pltpu.make_async_copy(
            x_hbm.at[pl.ds(row0 * N, GN)],
            xbuf.at[pl.ds(buf_i * GN, GN)], sem)

    copy_in(0, 0, sem0).start()

    def group_body(g, carry):
        parity = g & 1

        @pl.when(parity == 0)
        def _():
            copy_in(g, 0, sem0).wait()

        @pl.when(parity == 1)
        def _():
            copy_in(g, 1, sem1).wait()

        @pl.when((g + 1 < NGRP) & (parity == 0))
        def _():
            copy_in(g + 1, 1, sem1).start()

        @pl.when((g + 1 < NGRP) & (parity == 1))
        def _():
            copy_in(g + 1, 0, sem0).start()

        boff = parity * GN

        def key_body(j, c):
            p = j >> 2
            e0 = (j & 3) * 16
            for u in range(16):
                e = e0 + u
                a = xbuf[pl.ds(boff + (2 * p) * N + e * L, L)]
                b = xbuf[pl.ds(boff + (2 * p + 1) * N + e * L, L)]
                pk = plsc.pack(a, b, format=plsc.PackFormat.INTERLEAVED)
                kbuf[pl.ds(p * N + e * L, L)] = plsc.bitcast(pk, jnp.int32)
            return c

        lax.fori_loop(0, GROUP * N // (32 * L), key_body, 0)

        def pair_body(p, ovec):
            kbase = p * N

            def pk2bf(bitsA, bitsB):
                return plsc.bitcast(bitsA | (bitsB << 16), jnp.bfloat16)

            t1A = t1B = t2A = t2B = zero
            for i in range(NBITS):
                bitc = 32768 >> i
                c1A, c1B = t1A + bitc, t1B + bitc
                c2A, c2B = t2A + bitc, t2B + bitc
                cv1 = pk2bf(u2bits(c1A), u2bits(c1B))
                cv2 = pk2bf(u2bits(65535 - c2A), u2bits(65535 - c2B))

                def cbody(j, c, cv1=cv1, cv2=cv2):
                    c1a, c1b, c2a, c2b = c
                    for u in range(4):
                        v = plsc.bitcast(
                            kbuf[pl.ds(kbase + (j * 4 + u) * L, L)],
                            jnp.bfloat16)
                        i1 = jnp.where(v >= cv1, bone, bzero)
                        i2 = jnp.where(v <= cv2, bone, bzero)
                        if u % 2 == 0:
                            c1a = c1a + i1
                            c2a = c2a + i2
                        else:
                            c1b = c1b + i1
                            c2b = c2b + i2
                    return c1a, c1b, c2a, c2b

                c1a, c1b, c2a, c2b = lax.fori_loop(
                    0, N // (4 * L), cbody, (bzero, bzero, bzero, bzero))
                uA1, uB1 = plsc.unpack(c1a + c1b,
                                       format=plsc.PackFormat.INTERLEAVED)
                uA2, uB2 = plsc.unpack(c2a + c2b,
                                       format=plsc.PackFormat.INTERLEAVED)
                nA1, nB1 = allsum(uA1), allsum(uB1)
                nA2, nB2 = allsum(uA2), allsum(uB2)
                t1A = jnp.where(nA1 >= float(K), c1A, t1A)
                t1B = jnp.where(nB1 >= float(K), c1B, t1B)
                t2A = jnp.where(nA2 >= float(K), c2A, t2A)
                t2B = jnp.where(nB2 >= float(K), c2B, t2B)

            for rr, (t1, t2) in enumerate(((t1A, t2A), (t1B, t2B))):
                r = 2 * p + rr
                base = r * N
                bot = 65535 - t2

                val_top = 0.5 * (u2f32(t1) + u2f32(t1 + 15))
                val_bot = 0.5 * (u2f32(bot - 15) + u2f32(bot))
                ub = 0.5 * (u2f32(t1 + 15) + u2f32(t1 + 16))
                lb = 0.5 * (u2f32(bot - 16) + u2f32(bot - 15))

                def fbody(j, c, base=base, ub=ub, lb=lb):
                    cg, sg, cl, sl = c
                    for u in range(8):
                        xv = xbuf[pl.ds(boff + base + (j * 8 + u) * L, L)]
                        m1 = xv > ub
                        m2 = xv < lb
                        cg = cg + jnp.where(m1, one, zero)
                        sg = sg + jnp.where(m1, xv, fzero)
                        cl = cl + jnp.where(m2, one, zero)
                        sl = sl + jnp.where(m2, xv, fzero)
                    return cg, sg, cl, sl

                cg, sg, cl, sl = lax.fori_loop(
                    0, N // (8 * L), fbody, (zero, fzero, zero, fzero))

                ng = float(K) - allsum(cg.astype(jnp.float32))
                nl = float(K) - allsum(cl.astype(jnp.float32))
                sgv = allsum(sg)
                slv = allsum(sl)
                top_sum = sgv + ng * val_top
                bot_sum = slv + nl * val_bot
                outv = (top_sum * (1.0 / (2 * K))
                        + bot_sum * (ALPHA / (2 * K)))
                ovec = jnp.where(lanes == r, outv, ovec)
            return ovec

        ovec = lax.fori_loop(0, GROUP // 2, pair_body, fzero)
        outbuf[pl.ds(g * GROUP, GROUP)] = ovec
        return carry

    lax.fori_loop(0, NGRP, group_body, 0)
    pltpu.sync_copy(outbuf, out_hbm.at[pl.ds(wid * RPW, RPW)])


@jax.jit
def kernel(input):
    x = input.reshape(R * N)
    mesh = plsc.VectorSubcoreMesh(
        core_axis_name="c", subcore_axis_name="s",
        num_cores=NC, num_subcores=NS)
    out = pl.kernel(
        _kernel_body,
        out_type=jax.ShapeDtypeStruct((R,), jnp.float32),
        mesh=mesh,
        compiler_params=pltpu.CompilerParams(needs_layout_passes=False),
        scratch_types=[
            pltpu.VMEM((2 * GROUP * N,), jnp.float32),
            pltpu.VMEM((GROUP * N // 2,), jnp.int32),
            pltpu.VMEM((RPW,), jnp.float32),
            pltpu.SemaphoreType.DMA,
            pltpu.SemaphoreType.DMA,
        ],
    )(x)
    return out.reshape(B, C)
